# baseline (speedup 1.0000x reference)
import functools

import jax
import jax.numpy as jnp
from jax import lax
from jax.experimental import pallas as pl
from jax.experimental.pallas import tpu as pltpu
from jax.experimental.pallas import tpu_sc as plsc

B, N, C, M = 8, 65536, 128, 16384
NW = 32
WPB = NW // B
CPW = C // WPB
MPW = M // WPB
HALF = N // 2
NXP = 3 * N // HALF

_IOTA16 = functools.partial(lax.iota, jnp.int32, 16)


def _sc_body(xs_hbm, ys_hbm, zs_hbm, feat_hbm, inds_hbm,
             ox_hbm, oy_hbm, oz_hbm, out_feat_hbm,
             lists_v, buf0_v, buf1_v, fout0_v, stage_v,
             sem_in0, sem_in1):
    cid = lax.axis_index("c")
    sid = lax.axis_index("s")
    wid = sid * 2 + cid
    b = wid // WPB
    q = wid % WPB
    c0 = q * CPW
    bufs = (buf0_v, buf1_v)
    sems = (sem_in0, sem_in1)
    planes = (xs_hbm, ys_hbm, zs_hbm)
    outs = (ox_hbm, oy_hbm, oz_hbm)

    _scope_x = jax.named_scope("phase_x")
    _scope_x.__enter__()
    pltpu.sync_copy(inds_hbm.at[b, pl.ds(q * MPW, MPW)], stage_v)

    cps = [pltpu.async_copy(xs_hbm.at[b, pl.ds(0, HALF)], bufs[0], sems[0])]
    for t in range(2 * len(planes)):
        if t + 1 < 2 * len(planes):
            cps.append(pltpu.async_copy(
                planes[(t + 1) // 2].at[b, pl.ds(((t + 1) % 2) * HALF, HALF)],
                bufs[(t + 1) % 2], sems[(t + 1) % 2]))
        cps[t].wait()
        buf = bufs[t % 2]
        j, h = t // 2, t % 2

        @plsc.parallel_loop(0, MPW, 16, unroll=4)
        def _xpass(base, j=j, h=h, buf=buf):
            iv = stage_v[pl.ds(base, 16)]
            e = iv - h * HALF
            mask = (e >= 0) & (e < HALF)
            ec = jnp.clip(e, 0, HALF - 1)
            vals = plsc.load_gather(buf, [ec], mask=mask)
            plsc.store_scatter(fout0_v, [j * MPW + base + _IOTA16()],
                               vals, mask=mask)

    for j in range(len(planes)):
        pltpu.sync_copy(fout0_v.at[pl.ds(j * MPW, MPW)],
                        outs[j].at[b, pl.ds(q * MPW, MPW)])

    _scope_x.__exit__(None, None, None)
    _scope_b = jax.named_scope("phase_b")
    _scope_b.__enter__()
    pltpu.make_async_copy(
        feat_hbm.at[b, c0, pl.ds(0, HALF)], buf0_v, sem_in0).start()
    pltpu.make_async_copy(
        feat_hbm.at[b, c0, pl.ds(HALF, HALF)], buf1_v, sem_in1).start()

    lo_cur, hi_cur = 0, M
    for piece in range(M // MPW):
        pltpu.sync_copy(inds_hbm.at[b, pl.ds(piece * MPW, MPW)], stage_v)

        def _bucket(v, carry, piece=piece):
            lo, hi = carry
            base = pl.multiple_of(v * 16, 16)
            iv = stage_v[pl.ds(base, 16)]
            pos = piece * MPW + base + _IOTA16()
            mlo = iv < HALF
            pk_lo = lax.shift_left(pos, 16) | iv
            pk_hi = lax.shift_left(pos, 16) | (iv - HALF)
            nlo = jnp.sum(mlo.astype(jnp.int32))
            plsc.store_compressed(lists_v.at[pl.ds(lo, 16)], pk_lo, mask=mlo)
            nhi = 16 - nlo
            hi2 = hi - nhi
            plsc.store_compressed(lists_v.at[pl.ds(hi2, 16)], pk_hi,
                                  mask=jnp.logical_not(mlo))
            return lo + nlo, hi2

        lo_cur, hi_cur = lax.fori_loop(0, MPW // 16, _bucket,
                                       (lo_cur, hi_cur))
    lo_n = lo_cur

    _scope_b.__exit__(None, None, None)
    _scope_f = jax.named_scope("phase_f")
    _scope_f.__enter__()
    def _half_pass(buf, fout, s0, cnt):
        nfull = lax.shift_left(lax.shift_right_logical(cnt, 4), 4)

        @plsc.parallel_loop(0, nfull, 16, unroll=8)
        def _body(v):
            pk = lists_v[pl.ds(s0 + v, 16)]
            pos = lax.shift_right_logical(pk, 16)
            local = pk & (HALF - 1)
            vals = plsc.load_gather(buf, [local])
            plsc.store_scatter(fout, [pos], vals)

        ntail = lax.shift_right_logical((cnt - nfull) + 15, 4)

        def _tail(v, carry):
            base = s0 + nfull + v * 16
            mask = (nfull + v * 16 + _IOTA16()) < cnt
            pk = lists_v[pl.ds(base, 16)]
            pos = lax.shift_right_logical(pk, 16) & (M - 1)
            local = pk & (HALF - 1)
            vals = plsc.load_gather(buf, [local], mask=mask)
            plsc.store_scatter(fout, [pos], vals, mask=mask)
            return carry

        lax.fori_loop(0, ntail, _tail, 0)

    def _in_cp(r, half, buf, sem):
        return pltpu.make_async_copy(
            feat_hbm.at[b, c0 + r, pl.ds(half * HALF, HALF)], buf, sem)

    def _row(k, carry):
        knext = jnp.minimum(k + 1, CPW - 1)
        _in_cp(k, 0, buf0_v, sem_in0).wait()
        _half_pass(buf0_v, fout0_v, 0, lo_n)
        _in_cp(knext, 0, buf0_v, sem_in0).start()
        _in_cp(k, 1, buf1_v, sem_in1).wait()
        _half_pass(buf1_v, fout0_v, lo_n, M - lo_n)
        _in_cp(knext, 1, buf1_v, sem_in1).start()
        pltpu.sync_copy(fout0_v, out_feat_hbm.at[b, c0 + k])
        return carry

    lax.fori_loop(0, CPW, _row, 0)
    _in_cp(CPW - 1, 0, buf0_v, sem_in0).wait()
    _in_cp(CPW - 1, 1, buf1_v, sem_in1).wait()
    _scope_f.__exit__(None, None, None)


@jax.jit
def _sc_gather(xs, ys, zs, features, inds):
    mesh = plsc.VectorSubcoreMesh(core_axis_name="c", subcore_axis_name="s")
    kern = functools.partial(
        pl.kernel,
        mesh=mesh,
        compiler_params=pltpu.CompilerParams(needs_layout_passes=False),
        out_type=(
            jax.ShapeDtypeStruct((B, M), jnp.float32),
            jax.ShapeDtypeStruct((B, M), jnp.float32),
            jax.ShapeDtypeStruct((B, M), jnp.float32),
            jax.ShapeDtypeStruct((B, C, M), jnp.float32),
        ),
        scratch_types=[
            pltpu.VMEM((M + 16,), jnp.int32),
            pltpu.VMEM((HALF,), jnp.float32),
            pltpu.VMEM((HALF,), jnp.float32),
            pltpu.VMEM((M,), jnp.float32),
            pltpu.VMEM((MPW,), jnp.int32),
            pltpu.SemaphoreType.DMA,
            pltpu.SemaphoreType.DMA,
        ],
    )(_sc_body)
    return kern(xs, ys, zs, features, inds)


def kernel(xyz, features, sample_inds):
    inds = sample_inds.astype(jnp.int32)
    ox, oy, oz, out_feat = _sc_gather(
        xyz[:, :, 0], xyz[:, :, 1], xyz[:, :, 2], features, inds)
    new_xyz = jnp.stack([ox, oy, oz], axis=-1)
    return (new_xyz, out_feat, sample_inds)


# speedup vs baseline: 1.1108x; 1.1108x over previous
import functools

import jax
import jax.numpy as jnp
from jax import lax
from jax.experimental import pallas as pl
from jax.experimental.pallas import tpu as pltpu
from jax.experimental.pallas import tpu_sc as plsc

B, N, C, M = 8, 65536, 128, 16384
NW = 32
WPB = NW // B
CPW = C // WPB
MPW = M // WPB
HALF = N // 2
NXP = 3 * N // HALF

_IOTA16 = functools.partial(lax.iota, jnp.int32, 16)


def _sc_body(xs_hbm, ys_hbm, zs_hbm, feat_hbm, inds_hbm,
             ox_hbm, oy_hbm, oz_hbm, out_feat_hbm,
             lists_v, buf0_v, buf1_v, fout0_v, stage_v,
             sem_in0, sem_in1):
    cid = lax.axis_index("c")
    sid = lax.axis_index("s")
    wid = sid * 2 + cid
    b = wid // WPB
    q = wid % WPB
    c0 = q * CPW
    bufs = (buf0_v, buf1_v)
    sems = (sem_in0, sem_in1)
    planes = (xs_hbm, ys_hbm, zs_hbm)
    outs = (ox_hbm, oy_hbm, oz_hbm)

    pltpu.sync_copy(inds_hbm.at[b, pl.ds(q * MPW, MPW)], stage_v)

    cps = [pltpu.async_copy(xs_hbm.at[b, pl.ds(0, HALF)], bufs[0], sems[0])]
    for t in range(2 * len(planes)):
        if t + 1 < 2 * len(planes):
            cps.append(pltpu.async_copy(
                planes[(t + 1) // 2].at[b, pl.ds(((t + 1) % 2) * HALF, HALF)],
                bufs[(t + 1) % 2], sems[(t + 1) % 2]))
        cps[t].wait()
        buf = bufs[t % 2]
        j, h = t // 2, t % 2

        @plsc.parallel_loop(0, MPW, 16, unroll=4)
        def _xpass(base, j=j, h=h, buf=buf):
            iv = stage_v[pl.ds(base, 16)]
            e = iv - h * HALF
            mask = (e >= 0) & (e < HALF)
            ec = jnp.clip(e, 0, HALF - 1)
            vals = plsc.load_gather(buf, [ec], mask=mask)
            plsc.store_scatter(fout0_v, [j * MPW + base + _IOTA16()],
                               vals, mask=mask)

    for j in range(len(planes)):
        pltpu.sync_copy(fout0_v.at[pl.ds(j * MPW, MPW)],
                        outs[j].at[b, pl.ds(q * MPW, MPW)])

    pltpu.make_async_copy(
        feat_hbm.at[b, c0, pl.ds(0, HALF)], buf0_v, sem_in0).start()
    pltpu.make_async_copy(
        feat_hbm.at[b, c0, pl.ds(HALF, HALF)], buf1_v, sem_in1).start()

    lo_cur, hi_cur = jnp.int32(0), jnp.int32(M)
    for piece in range(M // MPW):
        pltpu.sync_copy(inds_hbm.at[b, pl.ds(piece * MPW, MPW)], stage_v)

        @plsc.parallel_loop(0, MPW, 16, unroll=4, carry=(lo_cur, hi_cur))
        def _bucket(base, carry, piece=piece):
            lo, hi = carry
            iv = stage_v[pl.ds(base, 16)]
            pos = piece * MPW + base + _IOTA16()
            mlo = iv < HALF
            pk_lo = lax.shift_left(pos, 16) | iv
            pk_hi = lax.shift_left(pos, 16) | (iv - HALF)
            nlo = jnp.sum(mlo.astype(jnp.int32))
            plsc.store_compressed(lists_v.at[pl.ds(lo, 16)], pk_lo, mask=mlo)
            nhi = 16 - nlo
            hi2 = hi - nhi
            plsc.store_compressed(lists_v.at[pl.ds(hi2, 16)], pk_hi,
                                  mask=jnp.logical_not(mlo))
            return lo + nlo, hi2

        lo_cur, hi_cur = _bucket
    lo_n = lo_cur

    def _half_pass(buf, fout, s0, cnt):
        nfull = lax.shift_left(lax.shift_right_logical(cnt, 4), 4)

        @plsc.parallel_loop(0, nfull, 16, unroll=8)
        def _body(v):
            pk = lists_v[pl.ds(s0 + v, 16)]
            pos = lax.shift_right_logical(pk, 16)
            local = pk & (HALF - 1)
            vals = plsc.load_gather(buf, [local])
            plsc.store_scatter(fout, [pos], vals)

        ntail = lax.shift_right_logical((cnt - nfull) + 15, 4)

        def _tail(v, carry):
            base = s0 + nfull + v * 16
            mask = (nfull + v * 16 + _IOTA16()) < cnt
            pk = lists_v[pl.ds(base, 16)]
            pos = lax.shift_right_logical(pk, 16) & (M - 1)
            local = pk & (HALF - 1)
            vals = plsc.load_gather(buf, [local], mask=mask)
            plsc.store_scatter(fout, [pos], vals, mask=mask)
            return carry

        lax.fori_loop(0, ntail, _tail, 0)

    def _in_cp(r, half, buf, sem):
        return pltpu.make_async_copy(
            feat_hbm.at[b, c0 + r, pl.ds(half * HALF, HALF)], buf, sem)

    def _row(k, carry):
        knext = jnp.minimum(k + 1, CPW - 1)
        _in_cp(k, 0, buf0_v, sem_in0).wait()
        _half_pass(buf0_v, fout0_v, 0, lo_n)
        _in_cp(knext, 0, buf0_v, sem_in0).start()
        _in_cp(k, 1, buf1_v, sem_in1).wait()
        _half_pass(buf1_v, fout0_v, lo_n, M - lo_n)
        _in_cp(knext, 1, buf1_v, sem_in1).start()
        pltpu.sync_copy(fout0_v, out_feat_hbm.at[b, c0 + k])
        return carry

    lax.fori_loop(0, CPW, _row, 0)
    _in_cp(CPW - 1, 0, buf0_v, sem_in0).wait()
    _in_cp(CPW - 1, 1, buf1_v, sem_in1).wait()


@jax.jit
def _sc_gather(xs, ys, zs, features, inds):
    mesh = plsc.VectorSubcoreMesh(core_axis_name="c", subcore_axis_name="s")
    kern = functools.partial(
        pl.kernel,
        mesh=mesh,
        compiler_params=pltpu.CompilerParams(needs_layout_passes=False),
        out_type=(
            jax.ShapeDtypeStruct((B, M), jnp.float32),
            jax.ShapeDtypeStruct((B, M), jnp.float32),
            jax.ShapeDtypeStruct((B, M), jnp.float32),
            jax.ShapeDtypeStruct((B, C, M), jnp.float32),
        ),
        scratch_types=[
            pltpu.VMEM((M + 16,), jnp.int32),
            pltpu.VMEM((HALF,), jnp.float32),
            pltpu.VMEM((HALF,), jnp.float32),
            pltpu.VMEM((M,), jnp.float32),
            pltpu.VMEM((MPW,), jnp.int32),
            pltpu.SemaphoreType.DMA,
            pltpu.SemaphoreType.DMA,
        ],
    )(_sc_body)
    return kern(xs, ys, zs, features, inds)


def kernel(xyz, features, sample_inds):
    inds = sample_inds.astype(jnp.int32)
    ox, oy, oz, out_feat = _sc_gather(
        xyz[:, :, 0], xyz[:, :, 1], xyz[:, :, 2], features, inds)
    new_xyz = jnp.stack([ox, oy, oz], axis=-1)
    return (new_xyz, out_feat, sample_inds)


# speedup vs baseline: 1.1143x; 1.0032x over previous
import functools

import jax
import jax.numpy as jnp
from jax import lax
from jax.experimental import pallas as pl
from jax.experimental.pallas import tpu as pltpu
from jax.experimental.pallas import tpu_sc as plsc

B, N, C, M = 8, 65536, 128, 16384
NW = 32
WPB = NW // B
CPW = C // WPB
MPW = M // WPB
HALF = N // 2

_IOTA16 = functools.partial(lax.iota, jnp.int32, 16)


def _sc_body(xs_hbm, ys_hbm, zs_hbm, feat_hbm, inds_hbm,
             ox_hbm, oy_hbm, oz_hbm, out_feat_hbm,
             lists_v, buf0_v, buf1_v, fout0_v, stage_v,
             sem_in0, sem_in1):
    cid = lax.axis_index("c")
    sid = lax.axis_index("s")
    wid = sid * 2 + cid
    b = wid // WPB
    q = wid % WPB
    c0 = q * CPW
    bufs = (buf0_v, buf1_v)
    sems = (sem_in0, sem_in1)
    planes = (xs_hbm, ys_hbm, zs_hbm)
    outs = (ox_hbm, oy_hbm, oz_hbm)

    pltpu.sync_copy(inds_hbm.at[b, pl.ds(q * MPW, MPW)], stage_v)

    cps = [pltpu.async_copy(xs_hbm.at[b, pl.ds(0, HALF)], bufs[0], sems[0])]
    for t in range(2 * len(planes)):
        if t + 1 < 2 * len(planes):
            cps.append(pltpu.async_copy(
                planes[(t + 1) // 2].at[b, pl.ds(((t + 1) % 2) * HALF, HALF)],
                bufs[(t + 1) % 2], sems[(t + 1) % 2]))
        cps[t].wait()
        buf = bufs[t % 2]
        j, h = t // 2, t % 2

        @plsc.parallel_loop(0, MPW, 16, unroll=4)
        def _xpass(base, j=j, h=h, buf=buf):
            iv = stage_v[pl.ds(base, 16)]
            e = iv - h * HALF
            mask = (e >= 0) & (e < HALF)
            ec = jnp.clip(e, 0, HALF - 1)
            vals = plsc.load_gather(buf, [ec], mask=mask)
            plsc.store_scatter(fout0_v, [j * MPW + base + _IOTA16()],
                               vals, mask=mask)

    for j in range(len(planes)):
        pltpu.sync_copy(fout0_v.at[pl.ds(j * MPW, MPW)],
                        outs[j].at[b, pl.ds(q * MPW, MPW)])

    pltpu.make_async_copy(
        feat_hbm.at[b, c0, pl.ds(0, HALF)], buf0_v, sem_in0).start()
    pltpu.make_async_copy(
        feat_hbm.at[b, c0, pl.ds(HALF, HALF)], buf1_v, sem_in1).start()

    lo_cur, hi_cur = jnp.int32(0), jnp.int32(M)
    for piece in range(M // MPW):
        pltpu.sync_copy(inds_hbm.at[b, pl.ds(piece * MPW, MPW)], stage_v)

        @plsc.parallel_loop(0, MPW, 16, unroll=4, carry=(lo_cur, hi_cur))
        def _bucket(base, carry, piece=piece):
            lo, hi = carry
            iv = stage_v[pl.ds(base, 16)]
            pos = piece * MPW + base + _IOTA16()
            mlo = iv < HALF
            pk_lo = lax.shift_left(pos, 16) | iv
            pk_hi = lax.shift_left(pos, 16) | (iv - HALF)
            nlo = jnp.sum(mlo.astype(jnp.int32))
            plsc.store_compressed(lists_v.at[pl.ds(lo, 16)], pk_lo, mask=mlo)
            nhi = 16 - nlo
            hi2 = hi - nhi
            plsc.store_compressed(lists_v.at[pl.ds(hi2, 16)], pk_hi,
                                  mask=jnp.logical_not(mlo))
            return lo + nlo, hi2

        lo_cur, hi_cur = _bucket
    lo_n = lo_cur

    def _half_pass(buf, fout, s0, cnt):
        nfull = lax.shift_left(lax.shift_right_logical(cnt, 4), 4)

        @plsc.parallel_loop(0, nfull, 16, unroll=8)
        def _body(v):
            pk = lists_v[pl.ds(s0 + v, 16)]
            pos = lax.shift_right_logical(pk, 16)
            local = pk & (HALF - 1)
            vals = plsc.load_gather(buf, [local])
            plsc.store_scatter(fout, [pos], vals)

        ntail = lax.shift_right_logical((cnt - nfull) + 15, 4)

        def _tail(v, carry):
            base = s0 + nfull + v * 16
            mask = (nfull + v * 16 + _IOTA16()) < cnt
            pk = lists_v[pl.ds(base, 16)]
            pos = lax.shift_right_logical(pk, 16) & (M - 1)
            local = pk & (HALF - 1)
            vals = plsc.load_gather(buf, [local], mask=mask)
            plsc.store_scatter(fout, [pos], vals, mask=mask)
            return carry

        lax.fori_loop(0, ntail, _tail, 0)

    def _in_cp(r, half, buf, sem):
        return pltpu.make_async_copy(
            feat_hbm.at[b, c0 + r, pl.ds(half * HALF, HALF)], buf, sem)

    def _row(k, carry):
        knext = jnp.minimum(k + 1, CPW - 1)
        _in_cp(k, 0, buf0_v, sem_in0).wait()
        _half_pass(buf0_v, fout0_v, 0, lo_n)
        _in_cp(knext, 0, buf0_v, sem_in0).start()
        _in_cp(k, 1, buf1_v, sem_in1).wait()
        _half_pass(buf1_v, fout0_v, lo_n, M - lo_n)
        _in_cp(knext, 1, buf1_v, sem_in1).start()
        pltpu.sync_copy(fout0_v, out_feat_hbm.at[b, c0 + k])
        return carry

    lax.fori_loop(0, CPW, _row, 0)
    _in_cp(CPW - 1, 0, buf0_v, sem_in0).wait()
    _in_cp(CPW - 1, 1, buf1_v, sem_in1).wait()


@jax.jit
def _sc_gather(xs, ys, zs, features, inds):
    mesh = plsc.VectorSubcoreMesh(core_axis_name="c", subcore_axis_name="s")
    kern = functools.partial(
        pl.kernel,
        mesh=mesh,
        compiler_params=pltpu.CompilerParams(needs_layout_passes=False),
        out_type=(
            jax.ShapeDtypeStruct((B, M), jnp.float32),
            jax.ShapeDtypeStruct((B, M), jnp.float32),
            jax.ShapeDtypeStruct((B, M), jnp.float32),
            jax.ShapeDtypeStruct((B, C, M), jnp.float32),
        ),
        scratch_types=[
            pltpu.VMEM((M + 16,), jnp.int32),
            pltpu.VMEM((HALF,), jnp.float32),
            pltpu.VMEM((HALF,), jnp.float32),
            pltpu.VMEM((M,), jnp.float32),
            pltpu.VMEM((MPW,), jnp.int32),
            pltpu.SemaphoreType.DMA,
            pltpu.SemaphoreType.DMA,
        ],
    )(_sc_body)
    return kern(xs, ys, zs, features, inds)


def kernel(xyz, features, sample_inds):
    inds = sample_inds.astype(jnp.int32)
    ox, oy, oz, out_feat = _sc_gather(
        xyz[:, :, 0], xyz[:, :, 1], xyz[:, :, 2], features, inds)
    new_xyz = jnp.stack([ox, oy, oz], axis=-1)
    return (new_xyz, out_feat, sample_inds)
